# Initial kernel scaffold; baseline (speedup 1.0000x reference)
import jax
import jax.numpy as jnp
from jax.experimental import pallas as pl


def kernel(relative_position_bias_table, relative_position_index):
    raise NotImplementedError("write your pallas kernel here")


# speedup vs baseline: 2.8078x; 2.8078x over previous
import jax
import jax.numpy as jnp
from jax import lax
from jax.experimental import pallas as pl
from jax.experimental.pallas import tpu as pltpu
from jax.experimental.pallas import tpu_sc as plsc

_NUM_CORES = 2
_NUM_SUBCORES = 16
_LANES = 16
_NUM_WORKERS = _NUM_CORES * _NUM_SUBCORES

_TABLE_ROWS = 961
_NUM_HEADS = 32
_N2 = 256 * 256
_POS_PER_WORKER = _N2 // _NUM_WORKERS
_GROUPS = _POS_PER_WORKER // _LANES


def _sc_body(table_hbm, idx_hbm, out_hbm, table_v, idx_v, out_v):
    wid = lax.axis_index("s") * _NUM_CORES + lax.axis_index("c")
    base = wid * _POS_PER_WORKER

    pltpu.sync_copy(table_hbm, table_v)
    pltpu.sync_copy(idx_hbm.at[pl.ds(base, _POS_PER_WORKER)], idx_v)

    def group(i, carry):
        off = i * _LANES
        ivec = idx_v[pl.ds(off, _LANES)]
        flat = ivec * _NUM_HEADS
        for h in range(_NUM_HEADS):
            vals = plsc.load_gather(table_v, [flat + h])
            out_v[pl.ds(h * _POS_PER_WORKER + off, _LANES)] = vals
        return carry

    lax.fori_loop(0, _GROUPS, group, 0)

    for h in range(_NUM_HEADS):
        pltpu.sync_copy(
            out_v.at[pl.ds(h * _POS_PER_WORKER, _POS_PER_WORKER)],
            out_hbm.at[pl.ds(h * _N2 + base, _POS_PER_WORKER)],
        )


def kernel(relative_position_bias_table, relative_position_index):
    table_flat = relative_position_bias_table.reshape(-1)
    idx_flat = relative_position_index.reshape(-1).astype(jnp.int32)

    mesh = plsc.VectorSubcoreMesh(
        core_axis_name="c",
        subcore_axis_name="s",
        num_cores=_NUM_CORES,
        num_subcores=_NUM_SUBCORES,
    )
    out_flat = pl.kernel(
        _sc_body,
        out_type=jax.ShapeDtypeStruct((_NUM_HEADS * _N2,), jnp.float32),
        mesh=mesh,
        scratch_types=[
            pltpu.VMEM((_TABLE_ROWS * _NUM_HEADS,), jnp.float32),
            pltpu.VMEM((_POS_PER_WORKER,), jnp.int32),
            pltpu.VMEM((_NUM_HEADS * _POS_PER_WORKER,), jnp.float32),
        ],
        compiler_params=pltpu.CompilerParams(needs_layout_passes=False),
        name="relative_position_bias_sc",
    )(table_flat, idx_flat)

    return out_flat.reshape(_NUM_HEADS, 256, 256)


# speedup vs baseline: 2.8400x; 1.0115x over previous
import jax
import jax.numpy as jnp
from jax import lax
from jax.experimental import pallas as pl
from jax.experimental.pallas import tpu as pltpu
from jax.experimental.pallas import tpu_sc as plsc

_NUM_CORES = 2
_NUM_SUBCORES = 16
_LANES = 16
_NUM_WORKERS = _NUM_CORES * _NUM_SUBCORES

_TABLE_ROWS = 961
_NUM_HEADS = 32
_N2 = 256 * 256
_POS_PER_WORKER = _N2 // _NUM_WORKERS
_GROUPS = _POS_PER_WORKER // _LANES


def _sc_body(table_hbm, idx_hbm, out_hbm, table_v, idx_v, out_v, sem_t, sem_i):
    wid = lax.axis_index("s") * _NUM_CORES + lax.axis_index("c")
    base = wid * _POS_PER_WORKER

    t_copy = pltpu.async_copy(table_hbm, table_v, sem_t)
    i_copy = pltpu.async_copy(idx_hbm.at[pl.ds(base, _POS_PER_WORKER)], idx_v, sem_i)
    i_copy.wait()
    t_copy.wait()

    def group(i, carry):
        off = i * _LANES
        ivec = idx_v[pl.ds(off, _LANES)]
        flat = ivec * _NUM_HEADS
        for h in range(_NUM_HEADS):
            vals = plsc.load_gather(table_v, [flat + h])
            out_v[h, pl.ds(off, _LANES)] = vals
        return carry

    lax.fori_loop(0, _GROUPS, group, 0)

    pltpu.sync_copy(out_v, out_hbm.at[:, pl.ds(base, _POS_PER_WORKER)])


def kernel(relative_position_bias_table, relative_position_index):
    table_flat = relative_position_bias_table.reshape(-1)
    idx_flat = relative_position_index.reshape(-1).astype(jnp.int32)

    mesh = plsc.VectorSubcoreMesh(
        core_axis_name="c",
        subcore_axis_name="s",
        num_cores=_NUM_CORES,
        num_subcores=_NUM_SUBCORES,
    )
    out_flat = pl.kernel(
        _sc_body,
        out_type=jax.ShapeDtypeStruct((_NUM_HEADS, _N2), jnp.float32),
        mesh=mesh,
        scratch_types=[
            pltpu.VMEM((_TABLE_ROWS * _NUM_HEADS,), jnp.float32),
            pltpu.VMEM((_POS_PER_WORKER,), jnp.int32),
            pltpu.VMEM((_NUM_HEADS, _POS_PER_WORKER), jnp.float32),
            pltpu.SemaphoreType.DMA,
            pltpu.SemaphoreType.DMA,
        ],
        compiler_params=pltpu.CompilerParams(needs_layout_passes=False),
        name="relative_position_bias_sc",
    )(table_flat, idx_flat)

    return out_flat.reshape(_NUM_HEADS, 256, 256)


# speedup vs baseline: 4.5314x; 1.5956x over previous
import jax
import jax.numpy as jnp
from jax import lax
from jax.experimental import pallas as pl
from jax.experimental.pallas import tpu as pltpu
from jax.experimental.pallas import tpu_sc as plsc

_NUM_CORES = 2
_NUM_SUBCORES = 16
_LANES = 16
_NUM_WORKERS = _NUM_CORES * _NUM_SUBCORES

_TABLE_ROWS = 961
_NUM_HEADS = 32
_N2 = 256 * 256
_POS_PER_WORKER = _N2 // _NUM_WORKERS
_GROUPS = _POS_PER_WORKER // _LANES


def _sc_body(table_hbm, idx_hbm, out_hbm, table_v, idx_v, out_v, sem_t, sem_i):
    wid = lax.axis_index("s") * _NUM_CORES + lax.axis_index("c")
    base = wid * _POS_PER_WORKER

    t_copy = pltpu.async_copy(table_hbm, table_v, sem_t)
    i_copy = pltpu.async_copy(idx_hbm.at[pl.ds(base, _POS_PER_WORKER)], idx_v, sem_i)
    i_copy.wait()
    t_copy.wait()

    def group(i, carry):
        off = i * _LANES
        ivec = idx_v[pl.ds(off, _LANES)]
        for h in range(_NUM_HEADS):
            vals = plsc.load_gather(table_v, [ivec + h * _TABLE_ROWS])
            out_v[h, pl.ds(off, _LANES)] = vals
        return carry

    lax.fori_loop(0, _GROUPS, group, 0)

    pltpu.sync_copy(out_v, out_hbm.at[:, pl.ds(base, _POS_PER_WORKER)])


def kernel(relative_position_bias_table, relative_position_index):
    table_flat = relative_position_bias_table.T.reshape(-1)
    idx_flat = relative_position_index.reshape(-1).astype(jnp.int32)

    mesh = plsc.VectorSubcoreMesh(
        core_axis_name="c",
        subcore_axis_name="s",
        num_cores=_NUM_CORES,
        num_subcores=_NUM_SUBCORES,
    )
    out_flat = pl.kernel(
        _sc_body,
        out_type=jax.ShapeDtypeStruct((_NUM_HEADS, _N2), jnp.float32),
        mesh=mesh,
        scratch_types=[
            pltpu.VMEM((_TABLE_ROWS * _NUM_HEADS,), jnp.float32),
            pltpu.VMEM((_POS_PER_WORKER,), jnp.int32),
            pltpu.VMEM((_NUM_HEADS, _POS_PER_WORKER), jnp.float32),
            pltpu.SemaphoreType.DMA,
            pltpu.SemaphoreType.DMA,
        ],
        compiler_params=pltpu.CompilerParams(needs_layout_passes=False),
        name="relative_position_bias_sc",
    )(table_flat, idx_flat)

    return out_flat.reshape(_NUM_HEADS, 256, 256)


# speedup vs baseline: 6.0514x; 1.3354x over previous
import jax
import jax.numpy as jnp
from jax import lax
from jax.experimental import pallas as pl
from jax.experimental.pallas import tpu as pltpu
from jax.experimental.pallas import tpu_sc as plsc

_NUM_CORES = 2
_NUM_SUBCORES = 16
_LANES = 16
_NUM_WORKERS = _NUM_CORES * _NUM_SUBCORES

_TABLE_ROWS = 961
_NUM_HEADS = 32
_N2 = 256 * 256
_POS_PER_WORKER = _N2 // _NUM_WORKERS
_GROUPS = _POS_PER_WORKER // _LANES


def _sc_body(table_hbm, idx_hbm, out_hbm, table_v, idx_v, out_v, sem_t, sem_i):
    wid = lax.axis_index("s") * _NUM_CORES + lax.axis_index("c")
    base = wid * _POS_PER_WORKER

    t_copy = pltpu.async_copy(table_hbm, table_v, sem_t)
    i_copy = pltpu.async_copy(idx_hbm.at[pl.ds(base, _POS_PER_WORKER)], idx_v, sem_i)
    i_copy.wait()
    t_copy.wait()

    @plsc.parallel_loop(0, _GROUPS, 1, unroll=2)
    def group(i):
        off = i * _LANES
        ivec = idx_v[pl.ds(off, _LANES)]
        for h in range(_NUM_HEADS):
            vals = plsc.load_gather(table_v, [ivec + h * _TABLE_ROWS])
            out_v[h, pl.ds(off, _LANES)] = vals

    pltpu.sync_copy(out_v, out_hbm.at[:, pl.ds(base, _POS_PER_WORKER)])


def kernel(relative_position_bias_table, relative_position_index):
    table_flat = relative_position_bias_table.T.reshape(-1)
    idx_flat = relative_position_index.reshape(-1).astype(jnp.int32)

    mesh = plsc.VectorSubcoreMesh(
        core_axis_name="c",
        subcore_axis_name="s",
        num_cores=_NUM_CORES,
        num_subcores=_NUM_SUBCORES,
    )
    out_flat = pl.kernel(
        _sc_body,
        out_type=jax.ShapeDtypeStruct((_NUM_HEADS, _N2), jnp.float32),
        mesh=mesh,
        scratch_types=[
            pltpu.VMEM((_TABLE_ROWS * _NUM_HEADS,), jnp.float32),
            pltpu.VMEM((_POS_PER_WORKER,), jnp.int32),
            pltpu.VMEM((_NUM_HEADS, _POS_PER_WORKER), jnp.float32),
            pltpu.SemaphoreType.DMA,
            pltpu.SemaphoreType.DMA,
        ],
        compiler_params=pltpu.CompilerParams(needs_layout_passes=False),
        name="relative_position_bias_sc",
    )(table_flat, idx_flat)

    return out_flat.reshape(_NUM_HEADS, 256, 256)
